# Initial kernel scaffold; baseline (speedup 1.0000x reference)
import jax
import jax.numpy as jnp
from jax.experimental import pallas as pl


def kernel(labels, train, table):
    raise NotImplementedError("write your pallas kernel here")


# speedup vs baseline: 1.0375x; 1.0375x over previous
import functools

import jax
import jax.numpy as jnp
from jax import lax
from jax.experimental import pallas as pl
from jax.experimental.pallas import tpu as pltpu
from jax.experimental.pallas import tpu_sc as plsc

HIDDEN = 64
CHUNK = 128
NBUF = 4


@functools.lru_cache(maxsize=None)
def _build(B, V):
    info = plsc.get_sparse_core_info()
    NC, NS = info.num_cores, info.num_subcores
    NW = NC * NS
    bpw = B // NW
    nch = bpw // CHUNK
    iters = nch // NBUF
    mesh = plsc.VectorSubcoreMesh(core_axis_name="c", subcore_axis_name="s")

    @functools.partial(
        pl.kernel,
        mesh=mesh,
        compiler_params=pltpu.CompilerParams(use_tc_tiling_on_sc=False),
        out_type=jax.ShapeDtypeStruct((B, HIDDEN), jnp.float32),
        scratch_types=(
            [pltpu.VMEM((nch, CHUNK), jnp.int32)]
            + [pltpu.VMEM((CHUNK, HIDDEN), jnp.float32) for _ in range(NBUF)]
            + [pltpu.SemaphoreType.DMA for _ in range(NBUF)]
        ),
    )
    def k(idx_hbm, table_hbm, out_hbm, idx_v, b0, b1, b2, b3, s0, s1, s2, s3):
        bufs = (b0, b1, b2, b3)
        sems = (s0, s1, s2, s3)
        wid = lax.axis_index("s") * NC + lax.axis_index("c")
        base = wid * bpw
        pltpu.sync_copy(idx_hbm.at[wid], idx_v)

        def body(o, carry):
            c0 = o * NBUF
            g = [
                pltpu.async_copy(table_hbm.at[idx_v.at[c0 + i]], bufs[i], sems[i])
                for i in range(NBUF)
            ]
            st = []
            for i in range(NBUF):
                g[i].wait()
                st.append(
                    pltpu.async_copy(
                        bufs[i],
                        out_hbm.at[pl.ds(base + (c0 + i) * CHUNK, CHUNK)],
                        sems[i],
                    )
                )
            for cp in st:
                cp.wait()
            return carry

        lax.fori_loop(0, iters, body, None)

    return k


def kernel(labels, train, table):
    Bt, L = labels.shape
    B = Bt * L
    info = plsc.get_sparse_core_info()
    NW = info.num_cores * info.num_subcores
    idx = labels.reshape(NW, (B // NW) // CHUNK, CHUNK)
    k = _build(B, table.shape[0])
    out = k(idx, table)
    return out.reshape(Bt, L * HIDDEN)


# speedup vs baseline: 1.0497x; 1.0118x over previous
import functools

import jax
import jax.numpy as jnp
from jax import lax
from jax.experimental import pallas as pl
from jax.experimental.pallas import tpu as pltpu
from jax.experimental.pallas import tpu_sc as plsc

HIDDEN = 64
CHUNK = 128
NBUF = 8


@functools.lru_cache(maxsize=None)
def _build(B, V):
    info = plsc.get_sparse_core_info()
    NC, NS = info.num_cores, info.num_subcores
    NW = NC * NS
    bpw = B // NW
    nch = bpw // CHUNK
    iters = nch // NBUF
    mesh = plsc.VectorSubcoreMesh(core_axis_name="c", subcore_axis_name="s")

    @functools.partial(
        pl.kernel,
        mesh=mesh,
        compiler_params=pltpu.CompilerParams(use_tc_tiling_on_sc=False),
        out_type=jax.ShapeDtypeStruct((B, HIDDEN), jnp.float32),
        scratch_types=(
            [pltpu.VMEM((nch, CHUNK), jnp.int32)]
            + [pltpu.VMEM((CHUNK, HIDDEN), jnp.float32) for _ in range(NBUF)]
            + [pltpu.SemaphoreType.DMA for _ in range(NBUF)]
        ),
    )
    def k(idx_hbm, table_hbm, out_hbm, idx_v, *rest):
        bufs = rest[:NBUF]
        sems = rest[NBUF:]
        wid = lax.axis_index("s") * NC + lax.axis_index("c")
        base = wid * bpw
        pltpu.sync_copy(idx_hbm.at[wid], idx_v)

        def body(o, carry):
            c0 = o * NBUF
            g = [
                pltpu.async_copy(table_hbm.at[idx_v.at[c0 + i]], bufs[i], sems[i])
                for i in range(NBUF)
            ]
            st = []
            for i in range(NBUF):
                g[i].wait()
                st.append(
                    pltpu.async_copy(
                        bufs[i],
                        out_hbm.at[pl.ds(base + (c0 + i) * CHUNK, CHUNK)],
                        sems[i],
                    )
                )
            for cp in st:
                cp.wait()
            return carry

        lax.fori_loop(0, iters, body, None)

    return k


def kernel(labels, train, table):
    Bt, L = labels.shape
    B = Bt * L
    info = plsc.get_sparse_core_info()
    NW = info.num_cores * info.num_subcores
    idx = labels.reshape(NW, (B // NW) // CHUNK, CHUNK)
    k = _build(B, table.shape[0])
    out = k(idx, table)
    return out.reshape(Bt, L * HIDDEN)
